# baseline (speedup 1.0000x reference)
import functools

import jax
import jax.numpy as jnp
from jax import lax
from jax.experimental import pallas as pl
from jax.experimental.pallas import tpu as pltpu
from jax.experimental.pallas import tpu_sc as plsc

B = 1024
S = 200
D = 128
K = 50

NC = 2
NS = 16
NW = NC * NS
CH = 128
NCHB = B // CH
NTASK = K * NCHB
TPW = -(-NTASK // NW)
NBUF = 4
L = 16

_mesh = plsc.VectorSubcoreMesh(core_axis_name="c", subcore_axis_name="s")


@functools.partial(
    pl.kernel,
    mesh=_mesh,
    out_type=jax.ShapeDtypeStruct((K * B, D), jnp.float32),
    scratch_types=[
        pltpu.VMEM((K * L,), jnp.int32),
        pltpu.VMEM((TPW, CH), jnp.int32),
        pltpu.VMEM((NBUF, CH, D), jnp.float32),
        pltpu.SemaphoreType.DMA,
        pltpu.SemaphoreType.DMA,
    ],
)
def _gather(x_hbm, idxr_hbm, out_hbm, idx_v, gidx, buf, gsem, wsem):
    wid = lax.axis_index("s") * NC + lax.axis_index("c")

    pltpu.sync_copy(idxr_hbm, idx_v)

    iota = lax.iota(jnp.int32, L)

    def task_id(i):
        return lax.rem(wid + i * NW, NTASK)

    def build_row(i):
        t = task_id(i)
        k = t // NCHB
        rep = idx_v[pl.ds(k * L, L)]
        b_base = lax.rem(t, NCHB) * CH
        for h in range(CH // L):
            bv = jnp.full((L,), b_base + h * L, jnp.int32) + iota
            gidx[i, pl.ds(h * L, L)] = bv * S + rep

    def gstart(i):
        build_row(i)
        return pltpu.async_copy(x_hbm.at[gidx.at[i]], buf.at[i % NBUF], gsem)

    def wtarget(i):
        t = task_id(i)
        row0 = (t // NCHB) * B + lax.rem(t, NCHB) * CH
        return out_hbm.at[pl.ds(row0, CH)]

    gh = [None] * TPW
    wh = [None] * TPW
    for i in range(NBUF):
        gh[i] = gstart(i)
    for i in range(TPW):
        gh[i].wait()
        wh[i] = pltpu.async_copy(buf.at[i % NBUF], wtarget(i), wsem)
        ni = i + NBUF
        if ni < TPW:
            wh[i].wait()
            gh[ni] = gstart(ni)
    for i in range(TPW - NBUF, TPW):
        wh[i].wait()


def kernel(x, index):
    x2d = x.reshape(B * S, D)
    idx_rep = jnp.repeat(index, L)
    out2d = _gather(x2d, idx_rep)
    return out2d.reshape(K, B, D).transpose(1, 0, 2)


# speedup vs baseline: 1.0135x; 1.0135x over previous
import functools

import jax
import jax.numpy as jnp
from jax import lax
from jax.experimental import pallas as pl
from jax.experimental.pallas import tpu as pltpu
from jax.experimental.pallas import tpu_sc as plsc

B = 1024
S = 200
D = 128
K = 50

NC = 2
NS = 16
NW = NC * NS
NWK = K // 2
CH = 128
NCHB = B // CH
NCHK = 2 * NCHB
NBUF = 4
L = 16

_mesh = plsc.VectorSubcoreMesh(core_axis_name="c", subcore_axis_name="s")


@functools.partial(
    pl.kernel,
    mesh=_mesh,
    out_type=jax.ShapeDtypeStruct((K * B, D), jnp.float32),
    scratch_types=[
        pltpu.VMEM((K * L,), jnp.int32),
        pltpu.VMEM((NCHK, CH), jnp.int32),
        pltpu.VMEM((NBUF, CH, D), jnp.float32),
        pltpu.SemaphoreType.DMA,
        pltpu.SemaphoreType.DMA,
    ],
)
def _gather(x_hbm, idxr_hbm, out_hbm, idx_v, gidx, buf, gsem, wsem):
    wid = lax.axis_index("s") * NC + lax.axis_index("c")

    @pl.when(wid < NWK)
    def _body():
        k0 = wid * 2
        pltpu.sync_copy(idxr_hbm, idx_v)

        iota = lax.iota(jnp.int32, L)

        def build_row(c):
            k = k0 + c // NCHB
            rep = idx_v[pl.ds(k * L, L)]
            b_base = (c % NCHB) * CH
            for h in range(CH // L):
                bv = jnp.full((L,), b_base + h * L, jnp.int32) + iota
                gidx[c, pl.ds(h * L, L)] = bv * S + rep

        def gstart(c):
            build_row(c)
            return pltpu.async_copy(
                x_hbm.at[gidx.at[c]], buf.at[c % NBUF], gsem)

        def wtarget(c):
            k = k0 + c // NCHB
            return out_hbm.at[pl.ds(k * B + (c % NCHB) * CH, CH)]

        gh = [None] * NCHK
        wh = [None] * NCHK
        for c in range(NBUF):
            gh[c] = gstart(c)
        for c in range(NCHK):
            gh[c].wait()
            wh[c] = pltpu.async_copy(buf.at[c % NBUF], wtarget(c), wsem)
            nc_ = c + NBUF
            if nc_ < NCHK:
                wh[c].wait()
                gh[nc_] = gstart(nc_)
        for c in range(NCHK - NBUF, NCHK):
            wh[c].wait()


def kernel(x, index):
    x2d = x.reshape(B * S, D)
    idx_rep = jnp.repeat(index, L)
    out2d = _gather(x2d, idx_rep)
    return out2d.reshape(K, B, D).transpose(1, 0, 2)
